# baseline (speedup 1.0000x reference)
import functools

import jax
import jax.numpy as jnp
from jax import lax
from jax.experimental import pallas as pl
from jax.experimental.pallas import tpu as pltpu
from jax.experimental.pallas import tpu_sc as plsc

N = 10000
E = 320000
D = 128
ET = 16
NPAD = 10240
NC, NS = 2, 16
NW = NC * NS
CH = 128
NCH = 80
EPW = NCH * CH
E_PAD = NW * EPW
RPW = NPAD // NS
NB = 5
_F32 = jnp.float32


def _mesh():
    return plsc.VectorSubcoreMesh(core_axis_name="c", subcore_axis_name="s")


def _deg_sc(dst_r):
    @functools.partial(
        pl.kernel,
        out_type=jax.ShapeDtypeStruct((NC, NPAD), _F32),
        mesh=_mesh(),
        scratch_types=[
            pltpu.VMEM((NCH, CH), jnp.int32),
            pltpu.VMEM((CH,), _F32),
            pltpu.VMEM((RPW,), _F32),
            pltpu.VMEM_SHARED((NPAD,), _F32),
            pltpu.SemaphoreType.DMA((NB,)),
        ],
    )
    def k(dst_hbm, out_hbm, didx, ones_v, zrow_v, acc_sh, ssem):
        c = lax.axis_index("c")
        s = lax.axis_index("s")
        wid = s * NC + c
        zero16 = jnp.zeros((16,), _F32)
        one16 = jnp.ones((16,), _F32)
        for j in range(RPW // 16):
            zrow_v[pl.ds(j * 16, 16)] = zero16
        for j in range(CH // 16):
            ones_v[pl.ds(j * 16, 16)] = one16
        pltpu.sync_copy(dst_hbm.at[wid], didx)
        pltpu.sync_copy(zrow_v, acc_sh.at[pl.ds(s * RPW, RPW)])
        plsc.subcore_barrier()

        @pl.loop(0, NCH, step=NB)
        def _(i):
            for b in range(NB):
                pltpu.async_copy(ones_v, acc_sh.at[didx.at[i + b]],
                                 ssem.at[b], add=True)
            for b in range(NB):
                pltpu.make_async_copy(ones_v, acc_sh.at[didx.at[i + b]],
                                      ssem.at[b]).wait()

        plsc.subcore_barrier()
        pltpu.sync_copy(acc_sh.at[pl.ds(s * RPW, RPW)],
                        out_hbm.at[c, pl.ds(s * RPW, RPW)])

    return k(dst_r)


def _segsum_sc(table, src_r, dst_r):
    @functools.partial(
        pl.kernel,
        out_type=jax.ShapeDtypeStruct((NC, NPAD, D), _F32),
        mesh=_mesh(),
        scratch_types=[
            pltpu.VMEM((NCH, CH), jnp.int32),
            pltpu.VMEM((NCH, CH), jnp.int32),
            pltpu.VMEM((CH, D), _F32),
            pltpu.VMEM_SHARED((NPAD, D), _F32),
        ],
    )
    def k(table_hbm, src_hbm, dst_hbm, out_hbm, sidx, didx, rows_v, acc_sh):
        c = lax.axis_index("c")
        s = lax.axis_index("s")
        wid = s * NC + c
        zero16 = jnp.zeros((16,), _F32)

        pltpu.sync_copy(src_hbm.at[wid], sidx)
        pltpu.sync_copy(dst_hbm.at[wid], didx)

        @pl.loop(0, CH)
        def _(j):
            for t in range(D // 16):
                rows_v[j, pl.ds(t * 16, 16)] = zero16

        for t in range(RPW // CH):
            pltpu.sync_copy(rows_v, acc_sh.at[pl.ds(s * RPW + t * CH, CH)])
        plsc.subcore_barrier()

        @pl.loop(0, NCH)
        def _(i):
            pltpu.sync_copy(table_hbm.at[sidx.at[i]], rows_v)
            pltpu.sync_copy(rows_v, acc_sh.at[didx.at[i]], add=True)

        plsc.subcore_barrier()
        pltpu.sync_copy(acc_sh.at[pl.ds(s * RPW, RPW)],
                        out_hbm.at[c, pl.ds(s * RPW, RPW)])

    return k(table, src_r, dst_r)


def _edge_mix_sc(p1, p2, src_r, dst_r):
    @functools.partial(
        pl.kernel,
        out_type=jax.ShapeDtypeStruct((E_PAD, ET), _F32),
        mesh=_mesh(),
        compiler_params=pltpu.CompilerParams(use_tc_tiling_on_sc=False),
        scratch_types=[
            pltpu.VMEM((NCH, CH), jnp.int32),
            pltpu.VMEM((NCH, CH), jnp.int32),
            pltpu.VMEM((NB, CH, ET), _F32),
            pltpu.VMEM((NB, CH, ET), _F32),
            pltpu.SemaphoreType.DMA((NB,)),
            pltpu.SemaphoreType.DMA((NB,)),
            pltpu.SemaphoreType.DMA((NB,)),
        ],
    )
    def k(p1_hbm, p2_hbm, src_hbm, dst_hbm, out_hbm, sidx, didx, a_v, b_v,
          gsem, hsem, osem):
        c = lax.axis_index("c")
        s = lax.axis_index("s")
        wid = s * NC + c
        base = wid * EPW
        pltpu.sync_copy(src_hbm.at[wid], sidx)
        pltpu.sync_copy(dst_hbm.at[wid], didx)

        for b in range(NB):
            pltpu.async_copy(p1_hbm.at[sidx.at[b]], a_v.at[b], gsem.at[b])
            pltpu.async_copy(p2_hbm.at[didx.at[b]], b_v.at[b], hsem.at[b])

        @pl.loop(0, NCH, step=NB)
        def _(i):
            for b in range(NB):
                pltpu.make_async_copy(p1_hbm.at[sidx.at[i + b]], a_v.at[b],
                                      gsem.at[b]).wait()
                pltpu.make_async_copy(p2_hbm.at[didx.at[i + b]], b_v.at[b],
                                      hsem.at[b]).wait()
                for j in range(CH):
                    a_v[b, j] = a_v[b, j] + b_v[b, j]
                pltpu.async_copy(
                    a_v.at[b], out_hbm.at[pl.ds(base + (i + b) * CH, CH)],
                    osem.at[b])
            for b in range(NB):
                pltpu.make_async_copy(
                    a_v.at[b], out_hbm.at[pl.ds(base + (i + b) * CH, CH)],
                    osem.at[b]).wait()

                @pl.when(i + NB + b < NCH)
                def _():
                    pltpu.async_copy(p1_hbm.at[sidx.at[i + NB + b]],
                                     a_v.at[b], gsem.at[b])
                    pltpu.async_copy(p2_hbm.at[didx.at[i + NB + b]],
                                     b_v.at[b], hsem.at[b])

    return k(p1, p2, src_r, dst_r)


_BM = 1024


def _dot(a, b):
    return lax.dot_general(a, b, (((1,), (0,)), ((), ())),
                           precision=lax.Precision.HIGHEST,
                           preferred_element_type=_F32)


def _mm_tc(x, w):
    k_dim = w.shape[1]

    def body(x_ref, w_ref, o_ref):
        o_ref[...] = _dot(x_ref[...], w_ref[...])

    return pl.pallas_call(
        body,
        grid=(NPAD // _BM,),
        in_specs=[pl.BlockSpec((_BM, D), lambda i: (i, 0)),
                  pl.BlockSpec((D, k_dim), lambda i: (0, 0))],
        out_specs=pl.BlockSpec((_BM, k_dim), lambda i: (i, 0)),
        out_shape=jax.ShapeDtypeStruct((NPAD, k_dim), _F32),
    )(x, w)


def _rscale_tc(d0, d1, s1):
    def body(d0_ref, d1_ref, s_ref, r_ref, o_ref):
        deg = jnp.maximum(d0_ref[...] + d1_ref[...], 1.0)
        r = lax.rsqrt(deg)
        r_ref[...] = r
        o_ref[...] = s_ref[...] * r

    return pl.pallas_call(
        body,
        grid=(NPAD // _BM,),
        in_specs=[pl.BlockSpec((_BM, 1), lambda i: (i, 0)),
                  pl.BlockSpec((_BM, 1), lambda i: (i, 0)),
                  pl.BlockSpec((_BM, D), lambda i: (i, 0))],
        out_specs=[pl.BlockSpec((_BM, 1), lambda i: (i, 0)),
                   pl.BlockSpec((_BM, D), lambda i: (i, 0))],
        out_shape=[jax.ShapeDtypeStruct((NPAD, 1), _F32),
                   jax.ShapeDtypeStruct((NPAD, D), _F32)],
    )(d0, d1, s1)


def _layer_mid_tc(q0, q1, r, b, w):
    def body(q0_ref, q1_ref, r_ref, b_ref, w_ref, o_ref):
        h = jnp.maximum((q0_ref[...] + q1_ref[...]) * r_ref[...] + b_ref[...],
                        0.0)
        o_ref[...] = _dot(h, w_ref[...]) * r_ref[...]

    return pl.pallas_call(
        body,
        grid=(NPAD // _BM,),
        in_specs=[pl.BlockSpec((_BM, D), lambda i: (i, 0)),
                  pl.BlockSpec((_BM, D), lambda i: (i, 0)),
                  pl.BlockSpec((_BM, 1), lambda i: (i, 0)),
                  pl.BlockSpec((1, D), lambda i: (0, 0)),
                  pl.BlockSpec((D, D), lambda i: (0, 0))],
        out_specs=pl.BlockSpec((_BM, D), lambda i: (i, 0)),
        out_shape=jax.ShapeDtypeStruct((NPAD, D), _F32),
    )(q0, q1, r, b, w)


def _layer_out_tc(q0, q1, r, b, wa, wb, bfc):
    def body(q0_ref, q1_ref, r_ref, b_ref, wa_ref, wb_ref, bfc_ref,
             p1_ref, p2_ref):
        h = jnp.maximum((q0_ref[...] + q1_ref[...]) * r_ref[...] + b_ref[...],
                        0.0)
        p1_ref[...] = _dot(h, wa_ref[...]) + bfc_ref[...]
        p2_ref[...] = _dot(h, wb_ref[...])

    return pl.pallas_call(
        body,
        grid=(NPAD // _BM,),
        in_specs=[pl.BlockSpec((_BM, D), lambda i: (i, 0)),
                  pl.BlockSpec((_BM, D), lambda i: (i, 0)),
                  pl.BlockSpec((_BM, 1), lambda i: (i, 0)),
                  pl.BlockSpec((1, D), lambda i: (0, 0)),
                  pl.BlockSpec((D, ET), lambda i: (0, 0)),
                  pl.BlockSpec((D, ET), lambda i: (0, 0)),
                  pl.BlockSpec((1, ET), lambda i: (0, 0))],
        out_specs=[pl.BlockSpec((_BM, ET), lambda i: (i, 0)),
                   pl.BlockSpec((_BM, ET), lambda i: (i, 0))],
        out_shape=[jax.ShapeDtypeStruct((NPAD, ET), _F32),
                   jax.ShapeDtypeStruct((NPAD, ET), _F32)],
    )(q0, q1, r, b, wa, wb, bfc)


def kernel(x, edges, W1, b1, W2, b2, Wfc, bfc):
    pad = E_PAD - E
    src_r = jnp.concatenate(
        [edges[0], jnp.zeros((pad,), jnp.int32)]).reshape(NW, NCH, CH)
    dst_r = jnp.concatenate(
        [edges[1], jnp.full((pad,), NPAD - 1, jnp.int32)]).reshape(NW, NCH, CH)
    xp = jnp.zeros((NPAD, D), _F32).at[:N].set(x)

    degp = _deg_sc(dst_r)
    s1 = _mm_tc(xp, W1)

    d0 = degp[0].reshape(NPAD, 1)
    d1 = degp[1].reshape(NPAD, 1)
    r, s1s = _rscale_tc(d0, d1, s1)

    qp = _segsum_sc(s1s, src_r, dst_r)
    s2s = _layer_mid_tc(qp[0], qp[1], r, b1.reshape(1, D), W2)

    qp2 = _segsum_sc(s2s, src_r, dst_r)
    p1, p2 = _layer_out_tc(qp2[0], qp2[1], r, b2.reshape(1, D),
                           Wfc[:D], Wfc[D:], bfc.reshape(1, ET))

    out_pad = _edge_mix_sc(p1, p2, src_r, dst_r)
    return out_pad[:E]


# speedup vs baseline: 1.0013x; 1.0013x over previous
import functools

import jax
import jax.numpy as jnp
from jax import lax
from jax.experimental import pallas as pl
from jax.experimental.pallas import tpu as pltpu
from jax.experimental.pallas import tpu_sc as plsc

N = 10000
E = 320000
D = 128
ET = 16
NPAD = 10240
NC, NS = 2, 16
NW = NC * NS
CH = 128
NCH = 80
EPW = NCH * CH
E_PAD = NW * EPW
RPW = NPAD // NS
NB = 5
_F32 = jnp.float32


def _mesh():
    return plsc.VectorSubcoreMesh(core_axis_name="c", subcore_axis_name="s")


def _deg_sc(dst_r):
    @functools.partial(
        pl.kernel,
        out_type=jax.ShapeDtypeStruct((NC, NPAD), _F32),
        mesh=_mesh(),
        scratch_types=[
            pltpu.VMEM((NCH, CH), jnp.int32),
            pltpu.VMEM((CH,), _F32),
            pltpu.VMEM((RPW,), _F32),
            pltpu.VMEM_SHARED((NPAD,), _F32),
            pltpu.SemaphoreType.DMA((NB,)),
        ],
    )
    def k(dst_hbm, out_hbm, didx, ones_v, zrow_v, acc_sh, ssem):
        c = lax.axis_index("c")
        s = lax.axis_index("s")
        wid = s * NC + c
        zero16 = jnp.zeros((16,), _F32)
        one16 = jnp.ones((16,), _F32)
        for j in range(RPW // 16):
            zrow_v[pl.ds(j * 16, 16)] = zero16
        for j in range(CH // 16):
            ones_v[pl.ds(j * 16, 16)] = one16
        pltpu.sync_copy(dst_hbm.at[wid], didx)
        pltpu.sync_copy(zrow_v, acc_sh.at[pl.ds(s * RPW, RPW)])
        plsc.subcore_barrier()

        @pl.loop(0, NCH, step=NB)
        def _(i):
            for b in range(NB):
                pltpu.async_copy(ones_v, acc_sh.at[didx.at[i + b]],
                                 ssem.at[b], add=True)
            for b in range(NB):
                pltpu.make_async_copy(ones_v, acc_sh.at[didx.at[i + b]],
                                      ssem.at[b]).wait()

        plsc.subcore_barrier()
        pltpu.sync_copy(acc_sh.at[pl.ds(s * RPW, RPW)],
                        out_hbm.at[c, pl.ds(s * RPW, RPW)])

    return k(dst_r)


def _segsum_sc(table, src_r, dst_r):
    @functools.partial(
        pl.kernel,
        out_type=jax.ShapeDtypeStruct((NC, NPAD, D), _F32),
        mesh=_mesh(),
        scratch_types=[
            pltpu.VMEM((NCH, CH), jnp.int32),
            pltpu.VMEM((NCH, CH), jnp.int32),
            pltpu.VMEM((CH, D), _F32),
            pltpu.VMEM_SHARED((NPAD, D), _F32),
        ],
    )
    def k(table_hbm, src_hbm, dst_hbm, out_hbm, sidx, didx, rows_v, acc_sh):
        c = lax.axis_index("c")
        s = lax.axis_index("s")
        wid = s * NC + c
        zero16 = jnp.zeros((16,), _F32)

        pltpu.sync_copy(src_hbm.at[wid], sidx)
        pltpu.sync_copy(dst_hbm.at[wid], didx)

        @pl.loop(0, CH)
        def _(j):
            for t in range(D // 16):
                rows_v[j, pl.ds(t * 16, 16)] = zero16

        for t in range(RPW // CH):
            pltpu.sync_copy(rows_v, acc_sh.at[pl.ds(s * RPW + t * CH, CH)])
        plsc.subcore_barrier()

        @pl.loop(0, NCH)
        def _(i):
            pltpu.sync_copy(table_hbm.at[sidx.at[i]], rows_v)
            pltpu.sync_copy(rows_v, acc_sh.at[didx.at[i]], add=True)

        plsc.subcore_barrier()
        pltpu.sync_copy(acc_sh.at[pl.ds(s * RPW, RPW)],
                        out_hbm.at[c, pl.ds(s * RPW, RPW)])

    return k(table, src_r, dst_r)


def _edge_mix_sc(p1, p2, src_r, dst_r):
    @functools.partial(
        pl.kernel,
        out_type=jax.ShapeDtypeStruct((E_PAD, ET), _F32),
        mesh=_mesh(),
        compiler_params=pltpu.CompilerParams(use_tc_tiling_on_sc=False),
        scratch_types=[
            pltpu.VMEM((NCH, CH), jnp.int32),
            pltpu.VMEM((NCH, CH), jnp.int32),
            pltpu.VMEM((NB, CH, ET), _F32),
            pltpu.VMEM((NB, CH, ET), _F32),
            pltpu.SemaphoreType.DMA((NB,)),
            pltpu.SemaphoreType.DMA((NB,)),
            pltpu.SemaphoreType.DMA((NB,)),
        ],
    )
    def k(p1_hbm, p2_hbm, src_hbm, dst_hbm, out_hbm, sidx, didx, a_v, b_v,
          gsem, hsem, osem):
        c = lax.axis_index("c")
        s = lax.axis_index("s")
        wid = s * NC + c
        base = wid * EPW
        pltpu.sync_copy(src_hbm.at[wid], sidx)
        pltpu.sync_copy(dst_hbm.at[wid], didx)

        for b in range(NB):
            pltpu.async_copy(p1_hbm.at[sidx.at[b]], a_v.at[b], gsem.at[b])
            pltpu.async_copy(p2_hbm.at[didx.at[b]], b_v.at[b], hsem.at[b])

        @pl.loop(0, NCH, step=NB)
        def _(i):
            for b in range(NB):
                pltpu.make_async_copy(p1_hbm.at[sidx.at[i + b]], a_v.at[b],
                                      gsem.at[b]).wait()
                pltpu.make_async_copy(p2_hbm.at[didx.at[i + b]], b_v.at[b],
                                      hsem.at[b]).wait()
                for j in range(CH):
                    a_v[b, j] = a_v[b, j] + b_v[b, j]
                pltpu.async_copy(
                    a_v.at[b], out_hbm.at[pl.ds(base + (i + b) * CH, CH)],
                    osem.at[b])
            for b in range(NB):
                pltpu.make_async_copy(
                    a_v.at[b], out_hbm.at[pl.ds(base + (i + b) * CH, CH)],
                    osem.at[b]).wait()

                @pl.when(i + NB + b < NCH)
                def _():
                    pltpu.async_copy(p1_hbm.at[sidx.at[i + NB + b]],
                                     a_v.at[b], gsem.at[b])
                    pltpu.async_copy(p2_hbm.at[didx.at[i + NB + b]],
                                     b_v.at[b], hsem.at[b])

    return k(p1, p2, src_r, dst_r)


_BM = 1024


def _dot(a, b):
    return lax.dot_general(a, b, (((1,), (0,)), ((), ())),
                           precision=lax.Precision.HIGHEST,
                           preferred_element_type=_F32)


def _mm_tc(x, w):
    k_dim = w.shape[1]

    def body(x_ref, w_ref, o_ref):
        o_ref[...] = _dot(x_ref[...], w_ref[...])

    return pl.pallas_call(
        body,
        grid=(NPAD // _BM,),
        in_specs=[pl.BlockSpec((_BM, D), lambda i: (i, 0)),
                  pl.BlockSpec((D, k_dim), lambda i: (0, 0))],
        out_specs=pl.BlockSpec((_BM, k_dim), lambda i: (i, 0)),
        out_shape=jax.ShapeDtypeStruct((NPAD, k_dim), _F32),
    )(x, w)


def _rscale_tc(d0, d1, s1):
    def body(d0_ref, d1_ref, s_ref, r_ref, o_ref):
        deg = jnp.maximum(d0_ref[...] + d1_ref[...], 1.0)
        r = lax.rsqrt(deg)
        r_ref[...] = r
        o_ref[...] = s_ref[...] * r

    return pl.pallas_call(
        body,
        grid=(NPAD // _BM,),
        in_specs=[pl.BlockSpec((_BM, 1), lambda i: (i, 0)),
                  pl.BlockSpec((_BM, 1), lambda i: (i, 0)),
                  pl.BlockSpec((_BM, D), lambda i: (i, 0))],
        out_specs=[pl.BlockSpec((_BM, 1), lambda i: (i, 0)),
                   pl.BlockSpec((_BM, D), lambda i: (i, 0))],
        out_shape=[jax.ShapeDtypeStruct((NPAD, 1), _F32),
                   jax.ShapeDtypeStruct((NPAD, D), _F32)],
    )(d0, d1, s1)


def _layer_mid_tc(q0, q1, r, b, w):
    def body(q0_ref, q1_ref, r_ref, b_ref, w_ref, o_ref):
        h = jnp.maximum((q0_ref[...] + q1_ref[...]) * r_ref[...] + b_ref[...],
                        0.0)
        o_ref[...] = _dot(h, w_ref[...]) * r_ref[...]

    return pl.pallas_call(
        body,
        grid=(NPAD // _BM,),
        in_specs=[pl.BlockSpec((_BM, D), lambda i: (i, 0)),
                  pl.BlockSpec((_BM, D), lambda i: (i, 0)),
                  pl.BlockSpec((_BM, 1), lambda i: (i, 0)),
                  pl.BlockSpec((1, D), lambda i: (0, 0)),
                  pl.BlockSpec((D, D), lambda i: (0, 0))],
        out_specs=pl.BlockSpec((_BM, D), lambda i: (i, 0)),
        out_shape=jax.ShapeDtypeStruct((NPAD, D), _F32),
    )(q0, q1, r, b, w)


def _layer_out_tc(q0, q1, r, b, wa, wb, bfc):
    def body(q0_ref, q1_ref, r_ref, b_ref, wa_ref, wb_ref, bfc_ref,
             p1_ref, p2_ref):
        h = jnp.maximum((q0_ref[...] + q1_ref[...]) * r_ref[...] + b_ref[...],
                        0.0)
        p1_ref[...] = _dot(h, wa_ref[...]) + bfc_ref[...]
        p2_ref[...] = _dot(h, wb_ref[...])

    return pl.pallas_call(
        body,
        grid=(NPAD // _BM,),
        in_specs=[pl.BlockSpec((_BM, D), lambda i: (i, 0)),
                  pl.BlockSpec((_BM, D), lambda i: (i, 0)),
                  pl.BlockSpec((_BM, 1), lambda i: (i, 0)),
                  pl.BlockSpec((1, D), lambda i: (0, 0)),
                  pl.BlockSpec((D, ET), lambda i: (0, 0)),
                  pl.BlockSpec((D, ET), lambda i: (0, 0)),
                  pl.BlockSpec((1, ET), lambda i: (0, 0))],
        out_specs=[pl.BlockSpec((_BM, ET), lambda i: (i, 0)),
                   pl.BlockSpec((_BM, ET), lambda i: (i, 0))],
        out_shape=[jax.ShapeDtypeStruct((NPAD, ET), _F32),
                   jax.ShapeDtypeStruct((NPAD, ET), _F32)],
    )(q0, q1, r, b, wa, wb, bfc)


def kernel(x, edges, W1, b1, W2, b2, Wfc, bfc):
    pad = E_PAD - E
    src_r = jnp.concatenate(
        [edges[0], jnp.zeros((pad,), jnp.int32)]).reshape(NW, NCH, CH)
    trash = N + (jnp.arange(pad, dtype=jnp.int32) % (NPAD - N))
    dst_r = jnp.concatenate([edges[1], trash]).reshape(NW, NCH, CH)
    xp = jnp.zeros((NPAD, D), _F32).at[:N].set(x)

    degp = _deg_sc(dst_r)
    s1 = _mm_tc(xp, W1)

    d0 = degp[0].reshape(NPAD, 1)
    d1 = degp[1].reshape(NPAD, 1)
    r, s1s = _rscale_tc(d0, d1, s1)

    qp = _segsum_sc(s1s, src_r, dst_r)
    s2s = _layer_mid_tc(qp[0], qp[1], r, b1.reshape(1, D), W2)

    qp2 = _segsum_sc(s2s, src_r, dst_r)
    p1, p2 = _layer_out_tc(qp2[0], qp2[1], r, b2.reshape(1, D),
                           Wfc[:D], Wfc[D:], bfc.reshape(1, ET))

    out_pad = _edge_mix_sc(p1, p2, src_r, dst_r)
    return out_pad[:E]


# speedup vs baseline: 1.2222x; 1.2205x over previous
import functools

import jax
import jax.numpy as jnp
from jax import lax
from jax.experimental import pallas as pl
from jax.experimental.pallas import tpu as pltpu
from jax.experimental.pallas import tpu_sc as plsc

N = 10000
E = 320000
D = 128
ET = 16
NPAD = 10240
NC, NS = 2, 16
CH = 128
PCH = 160
E_PAD = NS * PCH * CH
RPW = NPAD // NS
NB = 4
SEG0, SEG1 = 120, 40
MIX0, MIX1 = 96, 64
DEG0, DEG1 = 80, 80
_F32 = jnp.float32


def _mesh():
    return plsc.VectorSubcoreMesh(core_axis_name="c", subcore_axis_name="s")


def _splits(c, a, b):
    return [(0, a), (a, b)][c]


def _deg_sc(dst_r):
    @functools.partial(
        pl.kernel,
        out_type=jax.ShapeDtypeStruct((NC, NPAD), _F32),
        mesh=_mesh(),
        scratch_types=[
            pltpu.VMEM((max(DEG0, DEG1), CH), jnp.int32),
            pltpu.VMEM((CH,), _F32),
            pltpu.VMEM((RPW,), _F32),
            pltpu.VMEM_SHARED((NPAD,), _F32),
            pltpu.SemaphoreType.DMA((NB,)),
        ],
    )
    def k(dst_hbm, out_hbm, didx, ones_v, zrow_v, acc_sh, ssem):
        c = lax.axis_index("c")
        s = lax.axis_index("s")
        zero16 = jnp.zeros((16,), _F32)
        one16 = jnp.ones((16,), _F32)
        for j in range(RPW // 16):
            zrow_v[pl.ds(j * 16, 16)] = zero16
        for j in range(CH // 16):
            ones_v[pl.ds(j * 16, 16)] = one16
        pltpu.sync_copy(zrow_v, acc_sh.at[pl.ds(s * RPW, RPW)])

        for ci in range(NC):
            row0, nch = _splits(ci, DEG0, DEG1)

            @pl.when(c == ci)
            def _():
                pltpu.sync_copy(dst_hbm.at[s, pl.ds(row0, nch)],
                                didx.at[pl.ds(0, nch)])
                plsc.subcore_barrier()

                @pl.loop(0, nch, step=NB)
                def _(i):
                    for b in range(NB):
                        pltpu.async_copy(ones_v, acc_sh.at[didx.at[i + b]],
                                         ssem.at[b], add=True)
                    for b in range(NB):
                        pltpu.make_async_copy(
                            ones_v, acc_sh.at[didx.at[i + b]],
                            ssem.at[b]).wait()

        plsc.subcore_barrier()
        pltpu.sync_copy(acc_sh.at[pl.ds(s * RPW, RPW)],
                        out_hbm.at[c, pl.ds(s * RPW, RPW)])

    return k(dst_r)


def _segsum_sc(table, src_r, dst_r):
    @functools.partial(
        pl.kernel,
        out_type=jax.ShapeDtypeStruct((NC, NPAD, D), _F32),
        mesh=_mesh(),
        scratch_types=[
            pltpu.VMEM((max(SEG0, SEG1), CH), jnp.int32),
            pltpu.VMEM((max(SEG0, SEG1), CH), jnp.int32),
            pltpu.VMEM((CH, D), _F32),
            pltpu.VMEM_SHARED((NPAD, D), _F32),
        ],
    )
    def k(table_hbm, src_hbm, dst_hbm, out_hbm, sidx, didx, rows_v, acc_sh):
        c = lax.axis_index("c")
        s = lax.axis_index("s")
        zero16 = jnp.zeros((16,), _F32)

        @pl.loop(0, CH)
        def _(j):
            for t in range(D // 16):
                rows_v[j, pl.ds(t * 16, 16)] = zero16

        for t in range(RPW // CH):
            pltpu.sync_copy(rows_v, acc_sh.at[pl.ds(s * RPW + t * CH, CH)])

        for ci in range(NC):
            row0, nch = _splits(ci, SEG0, SEG1)

            @pl.when(c == ci)
            def _():
                pltpu.sync_copy(src_hbm.at[s, pl.ds(row0, nch)],
                                sidx.at[pl.ds(0, nch)])
                pltpu.sync_copy(dst_hbm.at[s, pl.ds(row0, nch)],
                                didx.at[pl.ds(0, nch)])
                plsc.subcore_barrier()

                @pl.loop(0, nch)
                def _(i):
                    pltpu.sync_copy(table_hbm.at[sidx.at[i]], rows_v)
                    pltpu.sync_copy(rows_v, acc_sh.at[didx.at[i]], add=True)

        plsc.subcore_barrier()
        pltpu.sync_copy(acc_sh.at[pl.ds(s * RPW, RPW)],
                        out_hbm.at[c, pl.ds(s * RPW, RPW)])

    return k(table, src_r, dst_r)


def _edge_mix_sc(p1, p2, src_r, dst_r):
    @functools.partial(
        pl.kernel,
        out_type=jax.ShapeDtypeStruct((E // 8, 128), _F32),
        mesh=_mesh(),
        compiler_params=pltpu.CompilerParams(use_tc_tiling_on_sc=False),
        scratch_types=[
            pltpu.VMEM((max(MIX0, MIX1), CH), jnp.int32),
            pltpu.VMEM((max(MIX0, MIX1), CH), jnp.int32),
            pltpu.VMEM((NB, CH, ET), _F32),
            pltpu.VMEM((NB, CH, ET), _F32),
            pltpu.VMEM((NB, CH // 8, 128), _F32),
            pltpu.SemaphoreType.DMA((NB,)),
            pltpu.SemaphoreType.DMA((NB,)),
            pltpu.SemaphoreType.DMA((NB,)),
        ],
    )
    def k(p1_hbm, p2_hbm, src_hbm, dst_hbm, out_hbm, sidx, didx, a_v, b_v,
          o_v, gsem, hsem, osem):
        c = lax.axis_index("c")
        s = lax.axis_index("s")

        for ci in range(NC):
            row0, nch = _splits(ci, MIX0, MIX1)

            @pl.when(c == ci)
            def _():
                pltpu.sync_copy(src_hbm.at[s, pl.ds(row0, nch)],
                                sidx.at[pl.ds(0, nch)])
                pltpu.sync_copy(dst_hbm.at[s, pl.ds(row0, nch)],
                                didx.at[pl.ds(0, nch)])
                cbase = s * PCH + row0

                for b in range(NB):
                    pltpu.async_copy(p1_hbm.at[sidx.at[b]], a_v.at[b],
                                     gsem.at[b])
                    pltpu.async_copy(p2_hbm.at[didx.at[b]], b_v.at[b],
                                     hsem.at[b])

                @pl.loop(0, nch, step=NB)
                def _(i):
                    for b in range(NB):
                        pltpu.make_async_copy(p1_hbm.at[sidx.at[i + b]],
                                              a_v.at[b], gsem.at[b]).wait()
                        pltpu.make_async_copy(p2_hbm.at[didx.at[i + b]],
                                              b_v.at[b], hsem.at[b]).wait()
                        for j in range(CH):
                            o_v[b, j // 8, pl.ds((j % 8) * ET, ET)] = (
                                a_v[b, j] + b_v[b, j])

                        @pl.when((cbase + i + b) * CH < E)
                        def _():
                            pltpu.async_copy(
                                o_v.at[b],
                                out_hbm.at[pl.ds(
                                    (cbase + i + b) * (CH // 8), CH // 8)],
                                osem.at[b])
                    for b in range(NB):
                        @pl.when((cbase + i + b) * CH < E)
                        def _():
                            pltpu.make_async_copy(
                                o_v.at[b],
                                out_hbm.at[pl.ds(
                                    (cbase + i + b) * (CH // 8), CH // 8)],
                                osem.at[b]).wait()

                        @pl.when(i + NB + b < nch)
                        def _():
                            pltpu.async_copy(p1_hbm.at[sidx.at[i + NB + b]],
                                             a_v.at[b], gsem.at[b])
                            pltpu.async_copy(p2_hbm.at[didx.at[i + NB + b]],
                                             b_v.at[b], hsem.at[b])

    return k(p1, p2, src_r, dst_r)


_BM = 1024


def _dot(a, b):
    return lax.dot_general(a, b, (((1,), (0,)), ((), ())),
                           precision=lax.Precision.HIGHEST,
                           preferred_element_type=_F32)


def _mm_tc(x, w):
    k_dim = w.shape[1]

    def body(x_ref, w_ref, o_ref):
        o_ref[...] = _dot(x_ref[...], w_ref[...])

    return pl.pallas_call(
        body,
        grid=(NPAD // _BM,),
        in_specs=[pl.BlockSpec((_BM, D), lambda i: (i, 0)),
                  pl.BlockSpec((D, k_dim), lambda i: (0, 0))],
        out_specs=pl.BlockSpec((_BM, k_dim), lambda i: (i, 0)),
        out_shape=jax.ShapeDtypeStruct((NPAD, k_dim), _F32),
    )(x, w)


def _rscale_tc(d0, d1, s1):
    def body(d0_ref, d1_ref, s_ref, r_ref, o_ref):
        deg = jnp.maximum(d0_ref[...] + d1_ref[...], 1.0)
        r = lax.rsqrt(deg)
        r_ref[...] = r
        o_ref[...] = s_ref[...] * r

    return pl.pallas_call(
        body,
        grid=(NPAD // _BM,),
        in_specs=[pl.BlockSpec((_BM, 1), lambda i: (i, 0)),
                  pl.BlockSpec((_BM, 1), lambda i: (i, 0)),
                  pl.BlockSpec((_BM, D), lambda i: (i, 0))],
        out_specs=[pl.BlockSpec((_BM, 1), lambda i: (i, 0)),
                   pl.BlockSpec((_BM, D), lambda i: (i, 0))],
        out_shape=[jax.ShapeDtypeStruct((NPAD, 1), _F32),
                   jax.ShapeDtypeStruct((NPAD, D), _F32)],
    )(d0, d1, s1)


def _layer_mid_tc(q0, q1, r, b, w):
    def body(q0_ref, q1_ref, r_ref, b_ref, w_ref, o_ref):
        h = jnp.maximum((q0_ref[...] + q1_ref[...]) * r_ref[...] + b_ref[...],
                        0.0)
        o_ref[...] = _dot(h, w_ref[...]) * r_ref[...]

    return pl.pallas_call(
        body,
        grid=(NPAD // _BM,),
        in_specs=[pl.BlockSpec((_BM, D), lambda i: (i, 0)),
                  pl.BlockSpec((_BM, D), lambda i: (i, 0)),
                  pl.BlockSpec((_BM, 1), lambda i: (i, 0)),
                  pl.BlockSpec((1, D), lambda i: (0, 0)),
                  pl.BlockSpec((D, D), lambda i: (0, 0))],
        out_specs=pl.BlockSpec((_BM, D), lambda i: (i, 0)),
        out_shape=jax.ShapeDtypeStruct((NPAD, D), _F32),
    )(q0, q1, r, b, w)


def _layer_out_tc(q0, q1, r, b, wa, wb, bfc):
    def body(q0_ref, q1_ref, r_ref, b_ref, wa_ref, wb_ref, bfc_ref,
             p1_ref, p2_ref):
        h = jnp.maximum((q0_ref[...] + q1_ref[...]) * r_ref[...] + b_ref[...],
                        0.0)
        p1_ref[...] = _dot(h, wa_ref[...]) + bfc_ref[...]
        p2_ref[...] = _dot(h, wb_ref[...])

    return pl.pallas_call(
        body,
        grid=(NPAD // _BM,),
        in_specs=[pl.BlockSpec((_BM, D), lambda i: (i, 0)),
                  pl.BlockSpec((_BM, D), lambda i: (i, 0)),
                  pl.BlockSpec((_BM, 1), lambda i: (i, 0)),
                  pl.BlockSpec((1, D), lambda i: (0, 0)),
                  pl.BlockSpec((D, ET), lambda i: (0, 0)),
                  pl.BlockSpec((D, ET), lambda i: (0, 0)),
                  pl.BlockSpec((1, ET), lambda i: (0, 0))],
        out_specs=[pl.BlockSpec((_BM, ET), lambda i: (i, 0)),
                   pl.BlockSpec((_BM, ET), lambda i: (i, 0))],
        out_shape=[jax.ShapeDtypeStruct((NPAD, ET), _F32),
                   jax.ShapeDtypeStruct((NPAD, ET), _F32)],
    )(q0, q1, r, b, wa, wb, bfc)


def kernel(x, edges, W1, b1, W2, b2, Wfc, bfc):
    pad = E_PAD - E
    src_r = jnp.concatenate(
        [edges[0], jnp.zeros((pad,), jnp.int32)]).reshape(NS, PCH, CH)
    trash = N + (jnp.arange(pad, dtype=jnp.int32) % (NPAD - N))
    dst_r = jnp.concatenate([edges[1], trash]).reshape(NS, PCH, CH)
    xp = jnp.zeros((NPAD, D), _F32).at[:N].set(x)

    degp = _deg_sc(dst_r)
    s1 = _mm_tc(xp, W1)

    d0 = degp[0].reshape(NPAD, 1)
    d1 = degp[1].reshape(NPAD, 1)
    r, s1s = _rscale_tc(d0, d1, s1)

    qp = _segsum_sc(s1s, src_r, dst_r)
    s2s = _layer_mid_tc(qp[0], qp[1], r, b1.reshape(1, D), W2)

    qp2 = _segsum_sc(s2s, src_r, dst_r)
    p1, p2 = _layer_out_tc(qp2[0], qp2[1], r, b2.reshape(1, D),
                           Wfc[:D], Wfc[D:], bfc.reshape(1, ET))

    out_packed = _edge_mix_sc(p1, p2, src_r, dst_r)
    return out_packed.reshape(E, ET)


# speedup vs baseline: 2.2352x; 1.8289x over previous
import functools

import jax
import jax.numpy as jnp
from jax import lax
from jax.experimental import pallas as pl
from jax.experimental.pallas import tpu as pltpu
from jax.experimental.pallas import tpu_sc as plsc

N = 10000
E = 320000
D = 128
ET = 16
NPAD = 10240
NC, NS = 2, 16
CH = 128
PCH = 160
E_PAD = NS * PCH * CH
RPW = NPAD // NS
NB = 4
SEG0, SEG1 = 80, 80
MIX0, MIX1 = 80, 80
DEG0, DEG1 = 80, 80
_F32 = jnp.float32


def _mesh():
    return plsc.VectorSubcoreMesh(core_axis_name="c", subcore_axis_name="s")


def _splits(c, a, b):
    return [(0, a), (a, b)][c]


def _deg_sc(dst_r):
    @functools.partial(
        pl.kernel,
        out_type=jax.ShapeDtypeStruct((NC, NPAD), _F32),
        mesh=_mesh(),
        scratch_types=[
            pltpu.VMEM((max(DEG0, DEG1), CH), jnp.int32),
            pltpu.VMEM((CH,), _F32),
            pltpu.VMEM((RPW,), _F32),
            pltpu.VMEM_SHARED((NPAD,), _F32),
            pltpu.SemaphoreType.DMA((NB,)),
        ],
    )
    def k(dst_hbm, out_hbm, didx, ones_v, zrow_v, acc_sh, ssem):
        c = lax.axis_index("c")
        s = lax.axis_index("s")
        zero16 = jnp.zeros((16,), _F32)
        one16 = jnp.ones((16,), _F32)
        for j in range(RPW // 16):
            zrow_v[pl.ds(j * 16, 16)] = zero16
        for j in range(CH // 16):
            ones_v[pl.ds(j * 16, 16)] = one16
        pltpu.sync_copy(zrow_v, acc_sh.at[pl.ds(s * RPW, RPW)])

        for ci in range(NC):
            row0, nch = _splits(ci, DEG0, DEG1)

            @pl.when(c == ci)
            def _():
                pltpu.sync_copy(dst_hbm.at[s, pl.ds(row0, nch)],
                                didx.at[pl.ds(0, nch)])
                plsc.subcore_barrier()

                @pl.loop(0, nch, step=NB)
                def _(i):
                    for b in range(NB):
                        pltpu.async_copy(ones_v, acc_sh.at[didx.at[i + b]],
                                         ssem.at[b], add=True)
                    for b in range(NB):
                        pltpu.make_async_copy(
                            ones_v, acc_sh.at[didx.at[i + b]],
                            ssem.at[b]).wait()

        plsc.subcore_barrier()
        pltpu.sync_copy(acc_sh.at[pl.ds(s * RPW, RPW)],
                        out_hbm.at[c, pl.ds(s * RPW, RPW)])

    return k(dst_r)


def _segsum_sc(table, src_r, dst_r):
    @functools.partial(
        pl.kernel,
        out_type=jax.ShapeDtypeStruct((NC, NPAD, D), _F32),
        mesh=_mesh(),
        scratch_types=[
            pltpu.VMEM((max(SEG0, SEG1), CH), jnp.int32),
            pltpu.VMEM((max(SEG0, SEG1), CH), jnp.int32),
            pltpu.VMEM((CH, D), _F32),
            pltpu.VMEM_SHARED((NPAD, D), _F32),
        ],
    )
    def k(table_hbm, src_hbm, dst_hbm, out_hbm, sidx, didx, rows_v, acc_sh):
        c = lax.axis_index("c")
        s = lax.axis_index("s")
        zero16 = jnp.zeros((16,), _F32)

        @pl.loop(0, CH)
        def _(j):
            for t in range(D // 16):
                rows_v[j, pl.ds(t * 16, 16)] = zero16

        for t in range(RPW // CH):
            pltpu.sync_copy(rows_v, acc_sh.at[pl.ds(s * RPW + t * CH, CH)])

        for ci in range(NC):
            row0, nch = _splits(ci, SEG0, SEG1)

            @pl.when(c == ci)
            def _():
                pltpu.sync_copy(src_hbm.at[s, pl.ds(row0, nch)],
                                sidx.at[pl.ds(0, nch)])
                pltpu.sync_copy(dst_hbm.at[s, pl.ds(row0, nch)],
                                didx.at[pl.ds(0, nch)])
                plsc.subcore_barrier()

                @pl.loop(0, nch)
                def _(i):
                    pltpu.sync_copy(table_hbm.at[sidx.at[i]], rows_v)
                    pltpu.sync_copy(rows_v, acc_sh.at[didx.at[i]], add=True)

        plsc.subcore_barrier()
        pltpu.sync_copy(acc_sh.at[pl.ds(s * RPW, RPW)],
                        out_hbm.at[c, pl.ds(s * RPW, RPW)])

    return k(table, src_r, dst_r)


def _edge_mix_sc(p1, p2, src_r, dst_r):
    @functools.partial(
        pl.kernel,
        out_type=jax.ShapeDtypeStruct((E // 8, 128), _F32),
        mesh=_mesh(),
        compiler_params=pltpu.CompilerParams(use_tc_tiling_on_sc=False),
        scratch_types=[
            pltpu.VMEM((max(MIX0, MIX1), CH), jnp.int32),
            pltpu.VMEM((max(MIX0, MIX1), CH), jnp.int32),
            pltpu.VMEM((NB, CH, ET), _F32),
            pltpu.VMEM((NB, CH, ET), _F32),
            pltpu.VMEM((NB, CH // 8, 128), _F32),
            pltpu.SemaphoreType.DMA((NB,)),
            pltpu.SemaphoreType.DMA((NB,)),
            pltpu.SemaphoreType.DMA((NB,)),
        ],
    )
    def k(p1_hbm, p2_hbm, src_hbm, dst_hbm, out_hbm, sidx, didx, a_v, b_v,
          o_v, gsem, hsem, osem):
        c = lax.axis_index("c")
        s = lax.axis_index("s")

        for ci in range(NC):
            row0, nch = _splits(ci, MIX0, MIX1)

            @pl.when(c == ci)
            def _():
                pltpu.sync_copy(src_hbm.at[s, pl.ds(row0, nch)],
                                sidx.at[pl.ds(0, nch)])
                pltpu.sync_copy(dst_hbm.at[s, pl.ds(row0, nch)],
                                didx.at[pl.ds(0, nch)])
                cbase = s * PCH + row0

                for b in range(NB):
                    pltpu.async_copy(p1_hbm.at[sidx.at[b]], a_v.at[b],
                                     gsem.at[b])
                    pltpu.async_copy(p2_hbm.at[didx.at[b]], b_v.at[b],
                                     hsem.at[b])

                @pl.loop(0, nch, step=NB)
                def _(i):
                    for b in range(NB):
                        pltpu.make_async_copy(p1_hbm.at[sidx.at[i + b]],
                                              a_v.at[b], gsem.at[b]).wait()
                        pltpu.make_async_copy(p2_hbm.at[didx.at[i + b]],
                                              b_v.at[b], hsem.at[b]).wait()
                        for j in range(CH):
                            o_v[b, j // 8, pl.ds((j % 8) * ET, ET)] = (
                                a_v[b, j] + b_v[b, j])

                        @pl.when((cbase + i + b) * CH < E)
                        def _():
                            pltpu.async_copy(
                                o_v.at[b],
                                out_hbm.at[pl.ds(
                                    (cbase + i + b) * (CH // 8), CH // 8)],
                                osem.at[b])
                    for b in range(NB):
                        @pl.when((cbase + i + b) * CH < E)
                        def _():
                            pltpu.make_async_copy(
                                o_v.at[b],
                                out_hbm.at[pl.ds(
                                    (cbase + i + b) * (CH // 8), CH // 8)],
                                osem.at[b]).wait()

                        @pl.when(i + NB + b < nch)
                        def _():
                            pltpu.async_copy(p1_hbm.at[sidx.at[i + NB + b]],
                                             a_v.at[b], gsem.at[b])
                            pltpu.async_copy(p2_hbm.at[didx.at[i + NB + b]],
                                             b_v.at[b], hsem.at[b])

    return k(p1, p2, src_r, dst_r)


_BM = 1024


def _dot(a, b):
    return lax.dot_general(a, b, (((1,), (0,)), ((), ())),
                           precision=lax.Precision.HIGHEST,
                           preferred_element_type=_F32)


def _mm_tc(x, w):
    k_dim = w.shape[1]

    def body(x_ref, w_ref, o_ref):
        o_ref[...] = _dot(x_ref[...], w_ref[...])

    return pl.pallas_call(
        body,
        grid=(NPAD // _BM,),
        in_specs=[pl.BlockSpec((_BM, D), lambda i: (i, 0)),
                  pl.BlockSpec((D, k_dim), lambda i: (0, 0))],
        out_specs=pl.BlockSpec((_BM, k_dim), lambda i: (i, 0)),
        out_shape=jax.ShapeDtypeStruct((NPAD, k_dim), _F32),
    )(x, w)


def _rscale_tc(d0, d1, s1):
    def body(d0_ref, d1_ref, s_ref, r_ref, o_ref):
        deg = jnp.maximum(d0_ref[...] + d1_ref[...], 1.0)
        r = lax.rsqrt(deg)
        r_ref[...] = r
        o_ref[...] = s_ref[...] * r

    return pl.pallas_call(
        body,
        grid=(NPAD // _BM,),
        in_specs=[pl.BlockSpec((_BM, 1), lambda i: (i, 0)),
                  pl.BlockSpec((_BM, 1), lambda i: (i, 0)),
                  pl.BlockSpec((_BM, D), lambda i: (i, 0))],
        out_specs=[pl.BlockSpec((_BM, 1), lambda i: (i, 0)),
                   pl.BlockSpec((_BM, D), lambda i: (i, 0))],
        out_shape=[jax.ShapeDtypeStruct((NPAD, 1), _F32),
                   jax.ShapeDtypeStruct((NPAD, D), _F32)],
    )(d0, d1, s1)


def _layer_mid_tc(q0, q1, r, b, w):
    def body(q0_ref, q1_ref, r_ref, b_ref, w_ref, o_ref):
        h = jnp.maximum((q0_ref[...] + q1_ref[...]) * r_ref[...] + b_ref[...],
                        0.0)
        o_ref[...] = _dot(h, w_ref[...]) * r_ref[...]

    return pl.pallas_call(
        body,
        grid=(NPAD // _BM,),
        in_specs=[pl.BlockSpec((_BM, D), lambda i: (i, 0)),
                  pl.BlockSpec((_BM, D), lambda i: (i, 0)),
                  pl.BlockSpec((_BM, 1), lambda i: (i, 0)),
                  pl.BlockSpec((1, D), lambda i: (0, 0)),
                  pl.BlockSpec((D, D), lambda i: (0, 0))],
        out_specs=pl.BlockSpec((_BM, D), lambda i: (i, 0)),
        out_shape=jax.ShapeDtypeStruct((NPAD, D), _F32),
    )(q0, q1, r, b, w)


def _layer_out_tc(q0, q1, r, b, wa, wb, bfc):
    def body(q0_ref, q1_ref, r_ref, b_ref, wa_ref, wb_ref, bfc_ref,
             p1_ref, p2_ref):
        h = jnp.maximum((q0_ref[...] + q1_ref[...]) * r_ref[...] + b_ref[...],
                        0.0)
        p1_ref[...] = _dot(h, wa_ref[...]) + bfc_ref[...]
        p2_ref[...] = _dot(h, wb_ref[...])

    return pl.pallas_call(
        body,
        grid=(NPAD // _BM,),
        in_specs=[pl.BlockSpec((_BM, D), lambda i: (i, 0)),
                  pl.BlockSpec((_BM, D), lambda i: (i, 0)),
                  pl.BlockSpec((_BM, 1), lambda i: (i, 0)),
                  pl.BlockSpec((1, D), lambda i: (0, 0)),
                  pl.BlockSpec((D, ET), lambda i: (0, 0)),
                  pl.BlockSpec((D, ET), lambda i: (0, 0)),
                  pl.BlockSpec((1, ET), lambda i: (0, 0))],
        out_specs=[pl.BlockSpec((_BM, ET), lambda i: (i, 0)),
                   pl.BlockSpec((_BM, ET), lambda i: (i, 0))],
        out_shape=[jax.ShapeDtypeStruct((NPAD, ET), _F32),
                   jax.ShapeDtypeStruct((NPAD, ET), _F32)],
    )(q0, q1, r, b, wa, wb, bfc)


def kernel(x, edges, W1, b1, W2, b2, Wfc, bfc):
    pad = E_PAD - E
    spread = jnp.arange(pad, dtype=jnp.int32) % N
    src_r = jnp.concatenate([edges[0], spread]).reshape(NS, PCH, CH)
    trash = N + (jnp.arange(pad, dtype=jnp.int32) % (NPAD - N))
    dst_r = jnp.concatenate([edges[1], trash]).reshape(NS, PCH, CH)
    xp = jnp.zeros((NPAD, D), _F32).at[:N].set(x)

    degp = _deg_sc(dst_r)
    s1 = _mm_tc(xp, W1)

    d0 = degp[0].reshape(NPAD, 1)
    d1 = degp[1].reshape(NPAD, 1)
    r, s1s = _rscale_tc(d0, d1, s1)

    qp = _segsum_sc(s1s, src_r, dst_r)
    s2s = _layer_mid_tc(qp[0], qp[1], r, b1.reshape(1, D), W2)

    qp2 = _segsum_sc(s2s, src_r, dst_r)
    p1, p2 = _layer_out_tc(qp2[0], qp2[1], r, b2.reshape(1, D),
                           Wfc[:D], Wfc[D:], bfc.reshape(1, ET))

    out_packed = _edge_mix_sc(p1, p2, src_r, dst_r)
    return out_packed.reshape(E, ET)


# speedup vs baseline: 2.2373x; 1.0009x over previous
import functools

import jax
import jax.numpy as jnp
from jax import lax
from jax.experimental import pallas as pl
from jax.experimental.pallas import tpu as pltpu
from jax.experimental.pallas import tpu_sc as plsc

N = 10000
E = 320000
D = 128
ET = 16
NPAD = 10240
NC, NS = 2, 16
CH = 128
PCH = 160
E_PAD = NS * PCH * CH
RPW = NPAD // NS
NB = 4
SEG0, SEG1 = 80, 80
MIX0, MIX1 = 80, 80
DEG0, DEG1 = 80, 80
_F32 = jnp.float32


def _mesh():
    return plsc.VectorSubcoreMesh(core_axis_name="c", subcore_axis_name="s")


def _splits(c, a, b):
    return [(0, a), (a, b)][c]


def _deg_sc(dst_r):
    @functools.partial(
        pl.kernel,
        out_type=jax.ShapeDtypeStruct((NC, NPAD), _F32),
        mesh=_mesh(),
        scratch_types=[
            pltpu.VMEM((max(DEG0, DEG1), CH), jnp.int32),
            pltpu.VMEM((CH,), _F32),
            pltpu.VMEM((RPW,), _F32),
            pltpu.VMEM_SHARED((NPAD,), _F32),
        ],
    )
    def k(dst_hbm, out_hbm, didx, ones_v, zrow_v, acc_sh):
        c = lax.axis_index("c")
        s = lax.axis_index("s")
        zero16 = jnp.zeros((16,), _F32)
        one16 = jnp.ones((16,), _F32)
        for j in range(RPW // 16):
            zrow_v[pl.ds(j * 16, 16)] = zero16
        for j in range(CH // 16):
            ones_v[pl.ds(j * 16, 16)] = one16
        pltpu.sync_copy(zrow_v, acc_sh.at[pl.ds(s * RPW, RPW)])

        for ci in range(NC):
            row0, nch = _splits(ci, DEG0, DEG1)

            @pl.when(c == ci)
            def _():
                pltpu.sync_copy(dst_hbm.at[s, pl.ds(row0, nch)],
                                didx.at[pl.ds(0, nch)])
                plsc.subcore_barrier()

                @pl.loop(0, nch)
                def _(i):
                    pltpu.sync_copy(ones_v, acc_sh.at[didx.at[i]], add=True)

        plsc.subcore_barrier()
        pltpu.sync_copy(acc_sh.at[pl.ds(s * RPW, RPW)],
                        out_hbm.at[c, pl.ds(s * RPW, RPW)])

    return k(dst_r)


def _segsum_sc(table, src_r, dst_r):
    @functools.partial(
        pl.kernel,
        out_type=jax.ShapeDtypeStruct((NC, NPAD, D), _F32),
        mesh=_mesh(),
        scratch_types=[
            pltpu.VMEM((max(SEG0, SEG1), CH), jnp.int32),
            pltpu.VMEM((max(SEG0, SEG1), CH), jnp.int32),
            pltpu.VMEM((CH, D), _F32),
            pltpu.VMEM_SHARED((NPAD, D), _F32),
        ],
    )
    def k(table_hbm, src_hbm, dst_hbm, out_hbm, sidx, didx, rows_v, acc_sh):
        c = lax.axis_index("c")
        s = lax.axis_index("s")
        zero16 = jnp.zeros((16,), _F32)

        @pl.loop(0, CH)
        def _(j):
            for t in range(D // 16):
                rows_v[j, pl.ds(t * 16, 16)] = zero16

        for t in range(RPW // CH):
            pltpu.sync_copy(rows_v, acc_sh.at[pl.ds(s * RPW + t * CH, CH)])

        for ci in range(NC):
            row0, nch = _splits(ci, SEG0, SEG1)

            @pl.when(c == ci)
            def _():
                pltpu.sync_copy(src_hbm.at[s, pl.ds(row0, nch)],
                                sidx.at[pl.ds(0, nch)])
                pltpu.sync_copy(dst_hbm.at[s, pl.ds(row0, nch)],
                                didx.at[pl.ds(0, nch)])
                plsc.subcore_barrier()

                @pl.loop(0, nch)
                def _(i):
                    pltpu.sync_copy(table_hbm.at[sidx.at[i]], rows_v)
                    pltpu.sync_copy(rows_v, acc_sh.at[didx.at[i]], add=True)

        plsc.subcore_barrier()
        pltpu.sync_copy(acc_sh.at[pl.ds(s * RPW, RPW)],
                        out_hbm.at[c, pl.ds(s * RPW, RPW)])

    return k(table, src_r, dst_r)


def _edge_mix_sc(p1, p2, src_r, dst_r):
    @functools.partial(
        pl.kernel,
        out_type=jax.ShapeDtypeStruct((E // 8, 128), _F32),
        mesh=_mesh(),
        compiler_params=pltpu.CompilerParams(use_tc_tiling_on_sc=False),
        scratch_types=[
            pltpu.VMEM((max(MIX0, MIX1), CH), jnp.int32),
            pltpu.VMEM((max(MIX0, MIX1), CH), jnp.int32),
            pltpu.VMEM((NB, CH, ET), _F32),
            pltpu.VMEM((NB, CH, ET), _F32),
            pltpu.VMEM((NB, CH // 8, 128), _F32),
            pltpu.SemaphoreType.DMA((NB,)),
            pltpu.SemaphoreType.DMA((NB,)),
            pltpu.SemaphoreType.DMA((NB,)),
        ],
    )
    def k(p1_hbm, p2_hbm, src_hbm, dst_hbm, out_hbm, sidx, didx, a_v, b_v,
          o_v, gsem, hsem, osem):
        c = lax.axis_index("c")
        s = lax.axis_index("s")

        for ci in range(NC):
            row0, nch = _splits(ci, MIX0, MIX1)

            @pl.when(c == ci)
            def _():
                pltpu.sync_copy(src_hbm.at[s, pl.ds(row0, nch)],
                                sidx.at[pl.ds(0, nch)])
                pltpu.sync_copy(dst_hbm.at[s, pl.ds(row0, nch)],
                                didx.at[pl.ds(0, nch)])
                cbase = s * PCH + row0

                for b in range(NB):
                    pltpu.async_copy(p1_hbm.at[sidx.at[b]], a_v.at[b],
                                     gsem.at[b])
                    pltpu.async_copy(p2_hbm.at[didx.at[b]], b_v.at[b],
                                     hsem.at[b])

                @pl.loop(0, nch, step=NB)
                def _(i):
                    for b in range(NB):
                        pltpu.make_async_copy(p1_hbm.at[sidx.at[i + b]],
                                              a_v.at[b], gsem.at[b]).wait()
                        pltpu.make_async_copy(p2_hbm.at[didx.at[i + b]],
                                              b_v.at[b], hsem.at[b]).wait()
                        for j in range(CH):
                            o_v[b, j // 8, pl.ds((j % 8) * ET, ET)] = (
                                a_v[b, j] + b_v[b, j])

                        @pl.when((cbase + i + b) * CH < E)
                        def _():
                            pltpu.async_copy(
                                o_v.at[b],
                                out_hbm.at[pl.ds(
                                    (cbase + i + b) * (CH // 8), CH // 8)],
                                osem.at[b])
                    for b in range(NB):
                        @pl.when((cbase + i + b) * CH < E)
                        def _():
                            pltpu.make_async_copy(
                                o_v.at[b],
                                out_hbm.at[pl.ds(
                                    (cbase + i + b) * (CH // 8), CH // 8)],
                                osem.at[b]).wait()

                        @pl.when(i + NB + b < nch)
                        def _():
                            pltpu.async_copy(p1_hbm.at[sidx.at[i + NB + b]],
                                             a_v.at[b], gsem.at[b])
                            pltpu.async_copy(p2_hbm.at[didx.at[i + NB + b]],
                                             b_v.at[b], hsem.at[b])

    return k(p1, p2, src_r, dst_r)


_BM = 1024


def _dot(a, b):
    return lax.dot_general(a, b, (((1,), (0,)), ((), ())),
                           precision=lax.Precision.HIGHEST,
                           preferred_element_type=_F32)


def _mm_tc(x, w):
    k_dim = w.shape[1]

    def body(x_ref, w_ref, o_ref):
        o_ref[...] = _dot(x_ref[...], w_ref[...])

    return pl.pallas_call(
        body,
        grid=(NPAD // _BM,),
        in_specs=[pl.BlockSpec((_BM, D), lambda i: (i, 0)),
                  pl.BlockSpec((D, k_dim), lambda i: (0, 0))],
        out_specs=pl.BlockSpec((_BM, k_dim), lambda i: (i, 0)),
        out_shape=jax.ShapeDtypeStruct((NPAD, k_dim), _F32),
    )(x, w)


def _rscale_tc(d0, d1, s1):
    def body(d0_ref, d1_ref, s_ref, r_ref, o_ref):
        deg = jnp.maximum(d0_ref[...] + d1_ref[...], 1.0)
        r = lax.rsqrt(deg)
        r_ref[...] = r
        o_ref[...] = s_ref[...] * r

    return pl.pallas_call(
        body,
        grid=(NPAD // _BM,),
        in_specs=[pl.BlockSpec((_BM, 1), lambda i: (i, 0)),
                  pl.BlockSpec((_BM, 1), lambda i: (i, 0)),
                  pl.BlockSpec((_BM, D), lambda i: (i, 0))],
        out_specs=[pl.BlockSpec((_BM, 1), lambda i: (i, 0)),
                   pl.BlockSpec((_BM, D), lambda i: (i, 0))],
        out_shape=[jax.ShapeDtypeStruct((NPAD, 1), _F32),
                   jax.ShapeDtypeStruct((NPAD, D), _F32)],
    )(d0, d1, s1)


def _layer_mid_tc(q0, q1, r, b, w):
    def body(q0_ref, q1_ref, r_ref, b_ref, w_ref, o_ref):
        h = jnp.maximum((q0_ref[...] + q1_ref[...]) * r_ref[...] + b_ref[...],
                        0.0)
        o_ref[...] = _dot(h, w_ref[...]) * r_ref[...]

    return pl.pallas_call(
        body,
        grid=(NPAD // _BM,),
        in_specs=[pl.BlockSpec((_BM, D), lambda i: (i, 0)),
                  pl.BlockSpec((_BM, D), lambda i: (i, 0)),
                  pl.BlockSpec((_BM, 1), lambda i: (i, 0)),
                  pl.BlockSpec((1, D), lambda i: (0, 0)),
                  pl.BlockSpec((D, D), lambda i: (0, 0))],
        out_specs=pl.BlockSpec((_BM, D), lambda i: (i, 0)),
        out_shape=jax.ShapeDtypeStruct((NPAD, D), _F32),
    )(q0, q1, r, b, w)


def _layer_out_tc(q0, q1, r, b, wa, wb, bfc):
    def body(q0_ref, q1_ref, r_ref, b_ref, wa_ref, wb_ref, bfc_ref,
             p1_ref, p2_ref):
        h = jnp.maximum((q0_ref[...] + q1_ref[...]) * r_ref[...] + b_ref[...],
                        0.0)
        p1_ref[...] = _dot(h, wa_ref[...]) + bfc_ref[...]
        p2_ref[...] = _dot(h, wb_ref[...])

    return pl.pallas_call(
        body,
        grid=(NPAD // _BM,),
        in_specs=[pl.BlockSpec((_BM, D), lambda i: (i, 0)),
                  pl.BlockSpec((_BM, D), lambda i: (i, 0)),
                  pl.BlockSpec((_BM, 1), lambda i: (i, 0)),
                  pl.BlockSpec((1, D), lambda i: (0, 0)),
                  pl.BlockSpec((D, ET), lambda i: (0, 0)),
                  pl.BlockSpec((D, ET), lambda i: (0, 0)),
                  pl.BlockSpec((1, ET), lambda i: (0, 0))],
        out_specs=[pl.BlockSpec((_BM, ET), lambda i: (i, 0)),
                   pl.BlockSpec((_BM, ET), lambda i: (i, 0))],
        out_shape=[jax.ShapeDtypeStruct((NPAD, ET), _F32),
                   jax.ShapeDtypeStruct((NPAD, ET), _F32)],
    )(q0, q1, r, b, wa, wb, bfc)


def kernel(x, edges, W1, b1, W2, b2, Wfc, bfc):
    pad = E_PAD - E
    spread = jnp.arange(pad, dtype=jnp.int32) % N
    src_r = jnp.concatenate([edges[0], spread]).reshape(NS, PCH, CH)
    trash = N + (jnp.arange(pad, dtype=jnp.int32) % (NPAD - N))
    dst_r = jnp.concatenate([edges[1], trash]).reshape(NS, PCH, CH)
    xp = jnp.zeros((NPAD, D), _F32).at[:N].set(x)

    degp = _deg_sc(dst_r)
    s1 = _mm_tc(xp, W1)

    d0 = degp[0].reshape(NPAD, 1)
    d1 = degp[1].reshape(NPAD, 1)
    r, s1s = _rscale_tc(d0, d1, s1)

    qp = _segsum_sc(s1s, src_r, dst_r)
    s2s = _layer_mid_tc(qp[0], qp[1], r, b1.reshape(1, D), W2)

    qp2 = _segsum_sc(s2s, src_r, dst_r)
    p1, p2 = _layer_out_tc(qp2[0], qp2[1], r, b2.reshape(1, D),
                           Wfc[:D], Wfc[D:], bfc.reshape(1, ET))

    out_packed = _edge_mix_sc(p1, p2, src_r, dst_r)
    return out_packed.reshape(E, ET)
